# Initial kernel scaffold; baseline (speedup 1.0000x reference)
import jax
import jax.numpy as jnp
from jax.experimental import pallas as pl


def kernel(x, W, roots, chamber_to_experts):
    raise NotImplementedError("write your pallas kernel here")


# speedup vs baseline: 1.3150x; 1.3150x over previous
import jax
import jax.numpy as jnp
from jax.experimental import pallas as pl

_BLOCK = 512


def _router_block(x_ref, wt_ref, roots_ref, tbl_ref, idx_ref, wts_ref):
    xb = x_ref[...].astype(jnp.bfloat16)
    wb = wt_ref[...].astype(jnp.bfloat16)
    h4 = jnp.dot(xb, wb, preferred_element_type=jnp.float32)
    nrm = jnp.sqrt(jnp.sum(h4 * h4, axis=1, keepdims=True))
    h4n = h4 / jnp.maximum(nrm, 1e-12)
    hb = h4n.astype(jnp.bfloat16).astype(jnp.float32)
    rb = roots_ref[...].astype(jnp.bfloat16).astype(jnp.float32)
    dots = jnp.concatenate(
        [jnp.sum(hb * rb[j, :], axis=1, keepdims=True) for j in range(4)],
        axis=1)
    pow2 = jnp.exp2(
        jax.lax.broadcasted_iota(jnp.int32, (_BLOCK, 4), 1).astype(jnp.float32))
    chamber = jnp.sum(jnp.where(dots >= 0.0, pow2, 0.0), axis=1, keepdims=True)
    iota16 = jax.lax.broadcasted_iota(
        jnp.int32, (_BLOCK, 16), 1).astype(jnp.float32)
    onehot = (chamber == iota16).astype(jnp.float32)
    pair = jnp.dot(onehot, tbl_ref[...].astype(jnp.float32),
                   preferred_element_type=jnp.float32)
    idx_ref[...] = pair.astype(jnp.int32)
    conf = jnp.min(jnp.abs(dots), axis=1, keepdims=True)
    w1 = 0.5 + 0.3 * jax.nn.sigmoid(conf)
    wts_ref[...] = jnp.concatenate([w1, 1.0 - w1], axis=1)


@jax.jit
def kernel(x, W, roots, chamber_to_experts):
    B, S, D = x.shape
    n = B * S
    x2 = x.reshape(n, D)
    grid = (n // _BLOCK,)
    idx, wts = pl.pallas_call(
        _router_block,
        grid=grid,
        in_specs=[
            pl.BlockSpec((_BLOCK, D), lambda i: (i, 0)),
            pl.BlockSpec((D, 4), lambda i: (0, 0)),
            pl.BlockSpec((4, 4), lambda i: (0, 0)),
            pl.BlockSpec((16, 2), lambda i: (0, 0)),
        ],
        out_specs=[
            pl.BlockSpec((_BLOCK, 2), lambda i: (i, 0)),
            pl.BlockSpec((_BLOCK, 2), lambda i: (i, 0)),
        ],
        out_shape=[
            jax.ShapeDtypeStruct((n, 2), jnp.int32),
            jax.ShapeDtypeStruct((n, 2), jnp.float32),
        ],
    )(x2, W.T, roots, chamber_to_experts)
    return idx.reshape(B, S, 2), wts.reshape(B, S, 2)


# speedup vs baseline: 1.3959x; 1.0616x over previous
import jax
import jax.numpy as jnp
from jax.experimental import pallas as pl

_BLOCK = 1024


def _router_block(x_ref, wt_ref, roots_ref, tbl_ref, idx_ref, wts_ref):
    xb = x_ref[...].astype(jnp.bfloat16)
    wb = wt_ref[...].astype(jnp.bfloat16)
    h4 = jnp.dot(xb, wb, preferred_element_type=jnp.float32)
    nrm = jnp.sqrt(jnp.sum(h4 * h4, axis=1, keepdims=True))
    h4n = h4 / jnp.maximum(nrm, 1e-12)
    hb = h4n.astype(jnp.bfloat16).astype(jnp.float32)
    rb = roots_ref[...].astype(jnp.bfloat16).astype(jnp.float32)
    dots = jnp.concatenate(
        [jnp.sum(hb * rb[j, :], axis=1, keepdims=True) for j in range(4)],
        axis=1)
    pow2 = jnp.exp2(
        jax.lax.broadcasted_iota(jnp.int32, (_BLOCK, 4), 1).astype(jnp.float32))
    chamber = jnp.sum(jnp.where(dots >= 0.0, pow2, 0.0), axis=1, keepdims=True)
    iota16 = jax.lax.broadcasted_iota(
        jnp.int32, (_BLOCK, 16), 1).astype(jnp.float32)
    onehot = (chamber == iota16).astype(jnp.float32)
    pair = jnp.dot(onehot, tbl_ref[...].astype(jnp.float32),
                   preferred_element_type=jnp.float32)
    idx_ref[...] = pair.astype(jnp.int32)
    conf = jnp.min(jnp.abs(dots), axis=1, keepdims=True)
    w1 = 0.5 + 0.3 * jax.nn.sigmoid(conf)
    wts_ref[...] = jnp.concatenate([w1, 1.0 - w1], axis=1)


@jax.jit
def kernel(x, W, roots, chamber_to_experts):
    B, S, D = x.shape
    n = B * S
    x2 = x.reshape(n, D)
    grid = (n // _BLOCK,)
    idx, wts = pl.pallas_call(
        _router_block,
        grid=grid,
        in_specs=[
            pl.BlockSpec((_BLOCK, D), lambda i: (i, 0)),
            pl.BlockSpec((D, 4), lambda i: (0, 0)),
            pl.BlockSpec((4, 4), lambda i: (0, 0)),
            pl.BlockSpec((16, 2), lambda i: (0, 0)),
        ],
        out_specs=[
            pl.BlockSpec((_BLOCK, 2), lambda i: (i, 0)),
            pl.BlockSpec((_BLOCK, 2), lambda i: (i, 0)),
        ],
        out_shape=[
            jax.ShapeDtypeStruct((n, 2), jnp.int32),
            jax.ShapeDtypeStruct((n, 2), jnp.float32),
        ],
    )(x2, W.T, roots, chamber_to_experts)
    return idx.reshape(B, S, 2), wts.reshape(B, S, 2)
